# baseline (speedup 1.0000x reference)
import functools

import jax
import jax.numpy as jnp
from jax import lax
from jax.experimental import pallas as pl
from jax.experimental.pallas import tpu as pltpu
from jax.experimental.pallas import tpu_sc as plsc

N = 16384
C = 5000
NUM_WORKERS = 16
PER_WORKER = N // NUM_WORKERS
WIN = 128
NUM_WIN = PER_WORKER // WIN
LANES = 16


def _loss_kernel(probT_hbm, tgt_hbm, rew_hbm, part_hbm, out_hbm,
                 tgt_v, rew_v, dst0_v, dst1_v, acc_v, all_v, out_v,
                 sem0, sem1):
    wid = lax.axis_index("s")
    base = wid * PER_WORKER

    pltpu.sync_copy(tgt_hbm.at[pl.ds(base, PER_WORKER)], tgt_v)
    pltpu.sync_copy(rew_hbm.at[pl.ds(base, PER_WORKER)], rew_v)

    dsts = (dst0_v, dst1_v)
    sems = (sem0, sem1)

    def fire(s):
        return pltpu.async_copy(
            probT_hbm.at[tgt_v.at[pl.ds(s * WIN, WIN)],
                         pl.ds(base + s * WIN, WIN)],
            dsts[s % 2], sems[s % 2])

    lane = lax.iota(jnp.int32, LANES)
    acc = jnp.zeros((LANES,), jnp.float32)
    cp = fire(0)
    for s in range(NUM_WIN):
        nxt = fire(s + 1) if s + 1 < NUM_WIN else None
        cp.wait()
        dst = dsts[s % 2]
        for g in range(WIN // LANES):
            kk = g * LANES + lane
            picked = plsc.load_gather(dst, [kk, kk])
            acc = acc + picked * rew_v[pl.ds(s * WIN + g * LANES, LANES)]
        cp = nxt
    acc_v[...] = acc

    pltpu.sync_copy(acc_v, part_hbm.at[wid])
    plsc.subcore_barrier()

    @pl.when(wid == 0)
    def _():
        pltpu.sync_copy(part_hbm, all_v)
        tot = jnp.zeros((LANES,), jnp.float32)
        for w in range(NUM_WORKERS):
            tot = tot + all_v[w]
        out_v[...] = -tot
        pltpu.sync_copy(out_v, out_hbm)


@jax.jit
def _loss(probT, target, reward):
    mesh = plsc.VectorSubcoreMesh(core_axis_name="c", subcore_axis_name="s",
                                  num_cores=1)
    k = functools.partial(
        pl.kernel,
        mesh=mesh,
        out_type=(jax.ShapeDtypeStruct((NUM_WORKERS, LANES), jnp.float32),
                  jax.ShapeDtypeStruct((LANES,), jnp.float32)),
        scratch_types=[
            pltpu.VMEM((PER_WORKER,), jnp.int32),
            pltpu.VMEM((PER_WORKER,), jnp.float32),
            pltpu.VMEM((WIN, WIN), jnp.float32),
            pltpu.VMEM((WIN, WIN), jnp.float32),
            pltpu.VMEM((LANES,), jnp.float32),
            pltpu.VMEM((NUM_WORKERS, LANES), jnp.float32),
            pltpu.VMEM((LANES,), jnp.float32),
            pltpu.SemaphoreType.DMA,
            pltpu.SemaphoreType.DMA,
        ],
        compiler_params=pltpu.CompilerParams(needs_layout_passes=False),
    )(_loss_kernel)
    return k(probT, target, reward)


def kernel(prob, target, reward):
    _, out16 = _loss(prob.T, target.astype(jnp.int32), reward)
    return jnp.sum(out16)


# speedup vs baseline: 1.1067x; 1.1067x over previous
import functools

import jax
import jax.numpy as jnp
from jax import lax
from jax.experimental import pallas as pl
from jax.experimental.pallas import tpu as pltpu
from jax.experimental.pallas import tpu_sc as plsc

N = 16384
C = 5000
NUM_CORES = 2
SUBCORES = 16
NUM_WORKERS = NUM_CORES * SUBCORES
PER_WORKER = N // NUM_WORKERS
WIN = 128
NUM_WIN = PER_WORKER // WIN
LANES = 16


def _loss_kernel(probT_hbm, tgt_hbm, rew_hbm, part_hbm, out_hbm,
                 tgt_v, rew_v, dst0_v, dst1_v, dst2_v, dst3_v,
                 acc_v, all_v, out_v, sem0, sem1, sem2, sem3):
    cid = lax.axis_index("c")
    sid = lax.axis_index("s")
    wid = cid * SUBCORES + sid
    base = wid * PER_WORKER

    pltpu.sync_copy(tgt_hbm.at[pl.ds(base, PER_WORKER)], tgt_v)
    pltpu.sync_copy(rew_hbm.at[pl.ds(base, PER_WORKER)], rew_v)

    dsts = (dst0_v, dst1_v, dst2_v, dst3_v)
    sems = (sem0, sem1, sem2, sem3)

    copies = [
        pltpu.async_copy(
            probT_hbm.at[tgt_v.at[pl.ds(s * WIN, WIN)],
                         pl.ds(base + s * WIN, WIN)],
            dsts[s], sems[s])
        for s in range(NUM_WIN)
    ]

    lane = lax.iota(jnp.int32, LANES)
    acc = jnp.zeros((LANES,), jnp.float32)
    for s in range(NUM_WIN):
        copies[s].wait()
        for g in range(WIN // LANES):
            kk = g * LANES + lane
            picked = plsc.load_gather(dsts[s], [kk, kk])
            acc = acc + picked * rew_v[pl.ds(s * WIN + g * LANES, LANES)]
    acc_v[...] = acc

    pltpu.sync_copy(acc_v, part_hbm.at[wid])
    plsc.subcore_barrier()

    @pl.when(sid == 0)
    def _():
        pltpu.sync_copy(part_hbm.at[pl.ds(cid * SUBCORES, SUBCORES)], all_v)
        tot = jnp.zeros((LANES,), jnp.float32)
        for w in range(SUBCORES):
            tot = tot + all_v[w]
        out_v[...] = -tot
        pltpu.sync_copy(out_v, out_hbm.at[cid])


@jax.jit
def _loss(probT, target, reward):
    mesh = plsc.VectorSubcoreMesh(core_axis_name="c", subcore_axis_name="s",
                                  num_cores=NUM_CORES)
    k = functools.partial(
        pl.kernel,
        mesh=mesh,
        out_type=(jax.ShapeDtypeStruct((NUM_WORKERS, LANES), jnp.float32),
                  jax.ShapeDtypeStruct((NUM_CORES, LANES), jnp.float32)),
        scratch_types=[
            pltpu.VMEM((PER_WORKER,), jnp.int32),
            pltpu.VMEM((PER_WORKER,), jnp.float32),
            pltpu.VMEM((WIN, WIN), jnp.float32),
            pltpu.VMEM((WIN, WIN), jnp.float32),
            pltpu.VMEM((WIN, WIN), jnp.float32),
            pltpu.VMEM((WIN, WIN), jnp.float32),
            pltpu.VMEM((LANES,), jnp.float32),
            pltpu.VMEM((SUBCORES, LANES), jnp.float32),
            pltpu.VMEM((LANES,), jnp.float32),
            pltpu.SemaphoreType.DMA,
            pltpu.SemaphoreType.DMA,
            pltpu.SemaphoreType.DMA,
            pltpu.SemaphoreType.DMA,
        ],
        compiler_params=pltpu.CompilerParams(needs_layout_passes=False),
    )(_loss_kernel)
    return k(probT, target, reward)


def kernel(prob, target, reward):
    _, out2 = _loss(prob.T, target.astype(jnp.int32), reward)
    return jnp.sum(out2)


# speedup vs baseline: 1.1557x; 1.0443x over previous
import functools

import jax
import jax.numpy as jnp
from jax import lax
from jax.experimental import pallas as pl
from jax.experimental.pallas import tpu as pltpu
from jax.experimental.pallas import tpu_sc as plsc

N = 16384
C = 5000
NUM_CORES = 2
SUBCORES = 16
NUM_WORKERS = NUM_CORES * SUBCORES
PER_WORKER = N // NUM_WORKERS
WIN = 128
NUM_WIN = PER_WORKER // WIN
LANES = 16


def _loss_kernel(probT_hbm, tgt_hbm, rew_hbm, part_hbm,
                 tgt_v, rew_v, dst0_v, dst1_v, dst2_v, dst3_v,
                 acc_v, sem0, sem1, sem2, sem3, semt, semr):
    cid = lax.axis_index("c")
    sid = lax.axis_index("s")
    wid = cid * SUBCORES + sid
    base = wid * PER_WORKER

    cpt = pltpu.async_copy(tgt_hbm.at[pl.ds(base, PER_WORKER)], tgt_v, semt)
    cpr = pltpu.async_copy(rew_hbm.at[pl.ds(base, PER_WORKER)], rew_v, semr)
    cpt.wait()

    dsts = (dst0_v, dst1_v, dst2_v, dst3_v)
    sems = (sem0, sem1, sem2, sem3)

    copies = [
        pltpu.async_copy(
            probT_hbm.at[tgt_v.at[pl.ds(s * WIN, WIN)],
                         pl.ds(base + s * WIN, WIN)],
            dsts[s], sems[s])
        for s in range(NUM_WIN)
    ]
    cpr.wait()

    lane = lax.iota(jnp.int32, LANES)
    acc = jnp.zeros((LANES,), jnp.float32)
    for s in range(NUM_WIN):
        copies[s].wait()
        for g in range(WIN // LANES):
            kk = g * LANES + lane
            picked = plsc.load_gather(dsts[s], [kk, kk])
            acc = acc + picked * rew_v[pl.ds(s * WIN + g * LANES, LANES)]
    acc_v[...] = -acc

    pltpu.sync_copy(acc_v, part_hbm.at[wid])


@jax.jit
def _loss(probT, target, reward):
    mesh = plsc.VectorSubcoreMesh(core_axis_name="c", subcore_axis_name="s",
                                  num_cores=NUM_CORES)
    k = functools.partial(
        pl.kernel,
        mesh=mesh,
        out_type=jax.ShapeDtypeStruct((NUM_WORKERS, LANES), jnp.float32),
        scratch_types=[
            pltpu.VMEM((PER_WORKER,), jnp.int32),
            pltpu.VMEM((PER_WORKER,), jnp.float32),
            pltpu.VMEM((WIN, WIN), jnp.float32),
            pltpu.VMEM((WIN, WIN), jnp.float32),
            pltpu.VMEM((WIN, WIN), jnp.float32),
            pltpu.VMEM((WIN, WIN), jnp.float32),
            pltpu.VMEM((LANES,), jnp.float32),
            pltpu.SemaphoreType.DMA,
            pltpu.SemaphoreType.DMA,
            pltpu.SemaphoreType.DMA,
            pltpu.SemaphoreType.DMA,
            pltpu.SemaphoreType.DMA,
            pltpu.SemaphoreType.DMA,
        ],
        compiler_params=pltpu.CompilerParams(needs_layout_passes=False),
    )(_loss_kernel)
    return k(probT, target, reward)


def kernel(prob, target, reward):
    part = _loss(prob.T, target.astype(jnp.int32), reward)
    return jnp.sum(part)


# speedup vs baseline: 1.1603x; 1.0040x over previous
import functools

import jax
import jax.numpy as jnp
from jax import lax
from jax.experimental import pallas as pl
from jax.experimental.pallas import tpu as pltpu
from jax.experimental.pallas import tpu_sc as plsc

N = 16384
C = 5000
NUM_CORES = 2
SUBCORES = 16
NUM_WORKERS = NUM_CORES * SUBCORES
PER_WORKER = N // NUM_WORKERS
WIN = 128
NUM_WIN = PER_WORKER // WIN
LANES = 16


def _loss_kernel(probT_hbm, tgt_hbm, rew_hbm, part_hbm,
                 tgt_v, rew_v, dst0_v, dst1_v, dst2_v, dst3_v,
                 acc_v, sem0, sem1, sem2, sem3, semt, semr):
    cid = lax.axis_index("c")
    sid = lax.axis_index("s")
    wid = cid * SUBCORES + sid
    base = wid * PER_WORKER

    cpt = pltpu.async_copy(tgt_hbm.at[pl.ds(base, PER_WORKER)], tgt_v, semt)
    cpr = pltpu.async_copy(rew_hbm.at[pl.ds(base, PER_WORKER)], rew_v, semr)
    cpt.wait()

    dsts = (dst0_v, dst1_v, dst2_v, dst3_v)
    sems = (sem0, sem1, sem2, sem3)

    copies = [
        pltpu.async_copy(
            probT_hbm.at[tgt_v.at[pl.ds(s * WIN, WIN)],
                         pl.ds(base + s * WIN, WIN)],
            dsts[s], sems[s])
        for s in range(NUM_WIN)
    ]
    cpr.wait()

    lane = lax.iota(jnp.int32, LANES)
    acc = jnp.zeros((LANES,), jnp.float32)
    for s in range(NUM_WIN):
        copies[s].wait()
        for g in range(WIN // LANES):
            kk = g * LANES + lane
            picked = plsc.load_gather(dsts[s], [kk, kk])
            acc = acc + picked * rew_v[pl.ds(s * WIN + g * LANES, LANES)]
    acc_v[...] = -acc

    pltpu.sync_copy(acc_v, part_hbm.at[wid])


@jax.jit
def _loss(probT, target, reward):
    mesh = plsc.VectorSubcoreMesh(core_axis_name="c", subcore_axis_name="s",
                                  num_cores=NUM_CORES)
    k = functools.partial(
        pl.kernel,
        mesh=mesh,
        out_type=jax.ShapeDtypeStruct((NUM_WORKERS, LANES), jnp.float32),
        scratch_types=[
            pltpu.VMEM((PER_WORKER,), jnp.int32),
            pltpu.VMEM((PER_WORKER,), jnp.float32),
            pltpu.VMEM((WIN, WIN), jnp.float32),
            pltpu.VMEM((WIN, WIN), jnp.float32),
            pltpu.VMEM((WIN, WIN), jnp.float32),
            pltpu.VMEM((WIN, WIN), jnp.float32),
            pltpu.VMEM((LANES,), jnp.float32),
            pltpu.SemaphoreType.DMA,
            pltpu.SemaphoreType.DMA,
            pltpu.SemaphoreType.DMA,
            pltpu.SemaphoreType.DMA,
            pltpu.SemaphoreType.DMA,
            pltpu.SemaphoreType.DMA,
        ],
        compiler_params=pltpu.CompilerParams(
            needs_layout_passes=False,
            skip_device_barrier=True,
        ),
    )(_loss_kernel)
    return k(probT, target, reward)


def kernel(prob, target, reward):
    part = _loss(prob.T, target.astype(jnp.int32), reward)
    return jnp.sum(part)
